# baseline (speedup 1.0000x reference)
import functools

import jax
import jax.numpy as jnp
from jax import lax
from jax.experimental import pallas as pl
from jax.experimental.pallas import tpu as pltpu
from jax.experimental.pallas import tpu_sc as plsc

_NC, _NS, _L = 2, 16, 16
_NW = _NC * _NS

_IDX_W = 128


@functools.lru_cache(maxsize=None)
def _build(B, F, V, D):
    rows_per_chunk = 64
    idxc = rows_per_chunk * F
    n_idx_rows = idxc // _IDX_W
    b_per_w = B // _NW
    chunks = b_per_w // rows_per_chunk
    assert idxc % _IDX_W == 0 and b_per_w % rows_per_chunk == 0
    assert _IDX_W % _L == 0

    w_idx_rows = chunks * n_idx_rows
    assert w_idx_rows % 8 == 0

    mesh = plsc.VectorSubcoreMesh(core_axis_name="c", subcore_axis_name="s")

    @functools.partial(
        pl.kernel,
        mesh=mesh,
        compiler_params=pltpu.CompilerParams(use_tc_tiling_on_sc=False),
        out_type=jax.ShapeDtypeStruct((B * F, D), jnp.float32),
        scratch_types=[
            pltpu.VMEM((w_idx_rows, _IDX_W), jnp.int32),
            pltpu.VMEM((n_idx_rows, _IDX_W), jnp.int32),
            pltpu.VMEM((2, idxc, D), jnp.float32),
            pltpu.SemaphoreType.DMA,
            pltpu.SemaphoreType.DMA,
        ],
    )
    def tok(idx_hbm, off_hbm, tab_hbm, out_hbm, idx_v, off_v, rows_v, gsem, osem):
        wid = lax.axis_index("s") * _NC + lax.axis_index("c")
        pltpu.sync_copy(off_hbm, off_v)
        pltpu.sync_copy(idx_hbm.at[pl.ds(wid * w_idx_rows, w_idx_rows)], idx_v)
        base_flat = wid * (chunks * idxc)

        def fire_out(c):
            return pltpu.async_copy(
                rows_v.at[c % 2], out_hbm.at[pl.ds(base_flat + c * idxc, idxc)],
                osem)

        gd = [None, None]
        od = [None, None]
        for c in range(chunks):
            b = c % 2
            if od[b] is not None:
                od[b].wait()
                od[b] = None
            for j in range(n_idx_rows):
                r = c * n_idx_rows + j
                for k in range(_IDX_W // _L):
                    s = pl.ds(k * _L, _L)
                    idx_v[r, s] = idx_v[r, s] + off_v[j, s]
            if c >= 1:
                pb = (c - 1) % 2
                for cp in gd[pb]:
                    cp.wait()
                gd[pb] = None
                od[pb] = fire_out(c - 1)
            gd[b] = [
                pltpu.async_copy(
                    tab_hbm.at[idx_v.at[c * n_idx_rows + j]],
                    rows_v.at[b, pl.ds(j * _IDX_W, _IDX_W)],
                    gsem,
                )
                for j in range(n_idx_rows)
            ]
        lb = (chunks - 1) % 2
        for cp in gd[lb]:
            cp.wait()
        od[lb] = fire_out(chunks - 1)
        for b in range(2):
            if od[b] is not None:
                od[b].wait()

    off = ((jnp.arange(n_idx_rows * _IDX_W, dtype=jnp.int32) % F) * V).reshape(
        n_idx_rows, _IDX_W)
    return tok, off


def kernel(indices, tables):
    B, F = indices.shape
    F2, V, D = tables.shape
    assert F2 == F
    tok, off = _build(B, F, V, D)
    idx2 = indices.astype(jnp.int32).reshape((B * F) // _IDX_W, _IDX_W)
    tab = tables.reshape(F * V, D)
    out = tok(idx2, off, tab)
    return out.reshape(B, F * D)


# speedup vs baseline: 1.0119x; 1.0119x over previous
import functools

import jax
import jax.numpy as jnp
from jax import lax
from jax.experimental import pallas as pl
from jax.experimental.pallas import tpu as pltpu
from jax.experimental.pallas import tpu_sc as plsc

_NC, _NS, _L = 2, 16, 16
_NW = _NC * _NS


@functools.lru_cache(maxsize=None)
def _build(B, F, V, D):
    rows_per_chunk = 64
    idxc = rows_per_chunk * F
    b_per_w = B // _NW
    chunks = b_per_w // rows_per_chunk
    assert idxc % _L == 0 and b_per_w % rows_per_chunk == 0
    assert chunks % 8 == 0
    assert idxc % 128 == 0

    mesh = plsc.VectorSubcoreMesh(core_axis_name="c", subcore_axis_name="s")

    @functools.partial(
        pl.kernel,
        mesh=mesh,
        compiler_params=pltpu.CompilerParams(use_tc_tiling_on_sc=False),
        out_type=jax.ShapeDtypeStruct((B * F, D), jnp.float32),
        scratch_types=[
            pltpu.VMEM((chunks, idxc), jnp.int32),
            pltpu.VMEM((idxc,), jnp.int32),
            pltpu.VMEM((2, idxc, D), jnp.float32),
            pltpu.SemaphoreType.DMA,
            pltpu.SemaphoreType.DMA,
        ],
    )
    def tok(idx_hbm, off_hbm, tab_hbm, out_hbm, idx_v, off_v, rows_v, gsem, osem):
        wid = lax.axis_index("s") * _NC + lax.axis_index("c")
        pltpu.sync_copy(off_hbm, off_v)
        pltpu.sync_copy(idx_hbm.at[pl.ds(wid * chunks, chunks)], idx_v)
        base_flat = wid * (chunks * idxc)

        def fire_out(c):
            return pltpu.async_copy(
                rows_v.at[c % 2], out_hbm.at[pl.ds(base_flat + c * idxc, idxc)],
                osem)

        gd = [None, None]
        od = [None, None]
        for c in range(chunks):
            b = c % 2
            if od[b] is not None:
                od[b].wait()
                od[b] = None
            for k in range(idxc // _L):
                s = pl.ds(k * _L, _L)
                idx_v[c, s] = idx_v[c, s] + off_v[s]
            if c >= 1:
                pb = (c - 1) % 2
                gd[pb].wait()
                gd[pb] = None
                od[pb] = fire_out(c - 1)
            gd[b] = pltpu.async_copy(
                tab_hbm.at[idx_v.at[c]], rows_v.at[b], gsem)
        lb = (chunks - 1) % 2
        gd[lb].wait()
        od[lb] = fire_out(chunks - 1)
        for b in range(2):
            if od[b] is not None:
                od[b].wait()

    off = (jnp.arange(idxc, dtype=jnp.int32) % F) * V
    return tok, off


def kernel(indices, tables):
    B, F = indices.shape
    F2, V, D = tables.shape
    assert F2 == F
    tok, off = _build(B, F, V, D)
    idxc = 64 * F
    idx2 = indices.astype(jnp.int32).reshape((B * F) // idxc, idxc)
    tab = tables.reshape(F * V, D)
    out = tok(idx2, off, tab)
    return out.reshape(B, F * D)
